# baseline (speedup 1.0000x reference)
import functools

import jax
import jax.numpy as jnp
from jax import lax
from jax.experimental import pallas as pl
from jax.experimental.pallas import tpu as pltpu
from jax.experimental.pallas import tpu_sc as plsc

_NUM_CORES = 2
_NUM_SUBCORES = 16
_NW = _NUM_CORES * _NUM_SUBCORES
_CHUNK = 128


def _sc_gather(user_ids, item_ids, U_gmf, I_gmf, U_mlp, I_mlp):
    n_rows, chunk = user_ids.shape
    d = U_gmf.shape[1]
    b = n_rows * chunk
    rows_per_w = n_rows // _NW
    b_per_w = rows_per_w * chunk

    mesh = plsc.VectorSubcoreMesh(
        core_axis_name="c", subcore_axis_name="s")
    out_type = [jax.ShapeDtypeStruct((b, d), jnp.float32)] * 4

    @functools.partial(
        pl.kernel,
        out_type=out_type,
        mesh=mesh,
        compiler_params=pltpu.CompilerParams(use_tc_tiling_on_sc=False),
        scratch_types=[
            pltpu.VMEM((rows_per_w, chunk), jnp.int32),
            pltpu.VMEM((rows_per_w, chunk), jnp.int32),
            pltpu.VMEM((b_per_w, d), jnp.float32),
            pltpu.VMEM((b_per_w, d), jnp.float32),
            pltpu.SemaphoreType.DMA,
            pltpu.SemaphoreType.DMA,
        ],
    )
    def gather_kernel(uidx_hbm, iidx_hbm, ug_hbm, ig_hbm, um_hbm, im_hbm,
                      ug_out, ig_out, um_out, im_out,
                      uidx_v, iidx_v, rows_a, rows_b, sem_a, sem_b):
        wid = lax.axis_index("s") * _NUM_CORES + lax.axis_index("c")
        row0 = wid * rows_per_w
        base = wid * b_per_w
        pltpu.sync_copy(uidx_hbm.at[pl.ds(row0, rows_per_w)], uidx_v)
        pltpu.sync_copy(iidx_hbm.at[pl.ds(row0, rows_per_w)], iidx_v)

        tables = ((ug_hbm, uidx_v, ug_out),
                  (ig_hbm, iidx_v, ig_out),
                  (um_hbm, uidx_v, um_out),
                  (im_hbm, iidx_v, im_out))

        def fire(t, buf, sem):
            tbl, idx_v, _ = tables[t]
            handles = []
            for c in range(rows_per_w):
                handles.append(pltpu.async_copy(
                    tbl.at[idx_v.at[c]],
                    buf.at[pl.ds(c * chunk, chunk)], sem))
            return handles

        def drain_and_store(t, buf, handles):
            for h in handles:
                h.wait()
            _, _, out = tables[t]
            pltpu.sync_copy(buf, out.at[pl.ds(base, b_per_w)])

        h_a = fire(0, rows_a, sem_a)
        h_b = fire(1, rows_b, sem_b)
        drain_and_store(0, rows_a, h_a)
        h_a = fire(2, rows_a, sem_a)
        drain_and_store(1, rows_b, h_b)
        h_b = fire(3, rows_b, sem_b)
        drain_and_store(2, rows_a, h_a)
        drain_and_store(3, rows_b, h_b)

    return gather_kernel(user_ids, item_ids, U_gmf, I_gmf, U_mlp, I_mlp)


def _mlp_body(ug, ig, um, im, w1u, w1i, b1, w2t, b2, wlg, wlm, bl, out):
    gmf = ug[:] * ig[:]
    h = (jnp.dot(um[:], w1u[:], preferred_element_type=jnp.float32)
         + jnp.dot(im[:], w1i[:], preferred_element_type=jnp.float32)
         + b1[:])
    h = jnp.maximum(h, 0.0)
    mlp = (jnp.dot(h, w2t[:], preferred_element_type=jnp.float32)
           + b2[:])
    z = (jnp.dot(gmf, wlg[:], preferred_element_type=jnp.float32)
         + jnp.dot(mlp, wlm[:], preferred_element_type=jnp.float32)
         + bl[:])
    out[:] = jax.nn.sigmoid(z)


def kernel(user_ids, item_ids, U_gmf, I_gmf, U_mlp, I_mlp,
           W1, b1, W2, b2, Wl, bl):
    b = user_ids.shape[0]
    d = U_gmf.shape[1]
    uidx = user_ids.astype(jnp.int32).reshape(b // _CHUNK, _CHUNK)
    iidx = item_ids.astype(jnp.int32).reshape(b // _CHUNK, _CHUNK)

    ug, ig, um, im = _sc_gather(uidx, iidx, U_gmf, I_gmf, U_mlp, I_mlp)

    w1u = W1[:, :d].T
    w1i = W1[:, d:].T
    w2t = W2.T
    wlg = Wl[0, :d].reshape(d, 1)
    wlm = Wl[0, d:].reshape(d, 1)
    b1r = b1.reshape(1, d)
    b2r = b2.reshape(1, d)
    blr = bl.reshape(1, 1)

    bb = 2048
    grid = (b // bb,)
    row_spec = pl.BlockSpec((bb, d), lambda i: (i, 0))
    full = lambda shape: pl.BlockSpec(shape, lambda i: (0, 0))

    return pl.pallas_call(
        _mlp_body,
        grid=grid,
        in_specs=[
            row_spec, row_spec, row_spec, row_spec,
            full((d, d)), full((d, d)), full((1, d)),
            full((d, d)), full((1, d)),
            full((d, 1)), full((d, 1)), full((1, 1)),
        ],
        out_specs=pl.BlockSpec((bb, 1), lambda i: (i, 0)),
        out_shape=jax.ShapeDtypeStruct((b, 1), jnp.float32),
    )(ug, ig, um, im, w1u, w1i, b1r, w2t, b2r, wlg, wlm, blr)


# speedup vs baseline: 1.0014x; 1.0014x over previous
import functools

import jax
import jax.numpy as jnp
from jax import lax
from jax.experimental import pallas as pl
from jax.experimental.pallas import tpu as pltpu
from jax.experimental.pallas import tpu_sc as plsc

_NUM_CORES = 2
_NUM_SUBCORES = 16
_NW = _NUM_CORES * _NUM_SUBCORES
_CHUNK = 128


def _sc_gather(uidx, iidx, ug_t, ig_t, um_t, im_t):
    n_rows, chunk = uidx.shape
    dd = ug_t.shape[1]
    b = n_rows * chunk
    rows_per_w = n_rows // _NW
    half = rows_per_w // 2
    b_stage = half * chunk

    mesh = plsc.VectorSubcoreMesh(
        core_axis_name="c", subcore_axis_name="s")
    out_type = [jax.ShapeDtypeStruct((b, dd), jnp.float32)] * 4

    @functools.partial(
        pl.kernel,
        out_type=out_type,
        mesh=mesh,
        scratch_types=[
            pltpu.VMEM((rows_per_w, chunk), jnp.int32),
            pltpu.VMEM((rows_per_w, chunk), jnp.int32),
            pltpu.VMEM((b_stage, dd), jnp.float32),
            pltpu.VMEM((b_stage, dd), jnp.float32),
            pltpu.SemaphoreType.DMA,
            pltpu.SemaphoreType.DMA,
        ],
    )
    def gather_kernel(uidx_hbm, iidx_hbm, ug_hbm, ig_hbm, um_hbm, im_hbm,
                      ug_out, ig_out, um_out, im_out,
                      uidx_v, iidx_v, rows_a, rows_b, sem_a, sem_b):
        wid = lax.axis_index("s") * _NUM_CORES + lax.axis_index("c")
        row0 = wid * rows_per_w
        base = wid * rows_per_w * chunk
        pltpu.sync_copy(uidx_hbm.at[pl.ds(row0, rows_per_w)], uidx_v)
        pltpu.sync_copy(iidx_hbm.at[pl.ds(row0, rows_per_w)], iidx_v)

        tables = ((ug_hbm, uidx_v, ug_out),
                  (ig_hbm, iidx_v, ig_out),
                  (um_hbm, uidx_v, um_out),
                  (im_hbm, iidx_v, im_out))
        bufs = (rows_a, rows_b)
        sems = (sem_a, sem_b)

        def fire(s):
            tbl, idx_v, _ = tables[s // 2]
            buf, sem = bufs[s % 2], sems[s % 2]
            h = s % 2
            handles = []
            for k in range(half):
                handles.append(pltpu.async_copy(
                    tbl.at[idx_v.at[h * half + k]],
                    buf.at[pl.ds(k * chunk, chunk)], sem))
            return handles

        def drain_store(s, handles):
            for hd in handles:
                hd.wait()
            _, _, out = tables[s // 2]
            h = s % 2
            pltpu.sync_copy(
                bufs[s % 2],
                out.at[pl.ds(base + h * b_stage, b_stage)])

        prev = fire(0)
        for s in range(1, 8):
            cur = fire(s)
            drain_store(s - 1, prev)
            prev = cur
        drain_store(7, prev)

    return gather_kernel(uidx, iidx, ug_t, ig_t, um_t, im_t)


def _mlp_body(up, ip, ug2, ig2, um2, im2,
              w1u, w1i, b1, w2t, b2, wlg, wlm, bl, out):
    def pick(x2, par):
        return jnp.where(par > 0, x2[:, 128 // 2:], x2[:, :128 // 2])

    ug = pick(ug2[:], up[:])
    ig = pick(ig2[:], ip[:])
    um = pick(um2[:], up[:])
    im = pick(im2[:], ip[:])
    gmf = ug * ig
    h = (jnp.dot(um, w1u[:], preferred_element_type=jnp.float32)
         + jnp.dot(im, w1i[:], preferred_element_type=jnp.float32)
         + b1[:])
    h = jnp.maximum(h, 0.0)
    mlp = (jnp.dot(h, w2t[:], preferred_element_type=jnp.float32)
           + b2[:])
    z = (jnp.dot(gmf, wlg[:], preferred_element_type=jnp.float32)
         + jnp.dot(mlp, wlm[:], preferred_element_type=jnp.float32)
         + bl[:])
    out[:] = jax.nn.sigmoid(z)


def kernel(user_ids, item_ids, U_gmf, I_gmf, U_mlp, I_mlp,
           W1, b1, W2, b2, Wl, bl):
    b = user_ids.shape[0]
    d = U_gmf.shape[1]
    uids = user_ids.astype(jnp.int32)
    iids = item_ids.astype(jnp.int32)
    uidx = (uids // 2).reshape(b // _CHUNK, _CHUNK)
    iidx = (iids // 2).reshape(b // _CHUNK, _CHUNK)
    up = (uids % 2).astype(jnp.float32).reshape(b, 1)
    ip = (iids % 2).astype(jnp.float32).reshape(b, 1)

    pack = lambda t: t.reshape(t.shape[0] // 2, 2 * d)
    ug2, ig2, um2, im2 = _sc_gather(
        uidx, iidx, pack(U_gmf), pack(I_gmf), pack(U_mlp), pack(I_mlp))

    w1u = W1[:, :d].T
    w1i = W1[:, d:].T
    w2t = W2.T
    wlg = Wl[0, :d].reshape(d, 1)
    wlm = Wl[0, d:].reshape(d, 1)
    b1r = b1.reshape(1, d)
    b2r = b2.reshape(1, d)
    blr = bl.reshape(1, 1)

    bb = 2048
    grid = (b // bb,)
    par_spec = pl.BlockSpec((bb, 1), lambda i: (i, 0))
    row_spec = pl.BlockSpec((bb, 2 * d), lambda i: (i, 0))
    full = lambda shape: pl.BlockSpec(shape, lambda i: (0, 0))

    return pl.pallas_call(
        _mlp_body,
        grid=grid,
        in_specs=[
            par_spec, par_spec,
            row_spec, row_spec, row_spec, row_spec,
            full((d, d)), full((d, d)), full((1, d)),
            full((d, d)), full((1, d)),
            full((d, 1)), full((d, 1)), full((1, 1)),
        ],
        out_specs=pl.BlockSpec((bb, 1), lambda i: (i, 0)),
        out_shape=jax.ShapeDtypeStruct((b, 1), jnp.float32),
    )(up, ip, ug2, ig2, um2, im2, w1u, w1i, b1r, w2t, b2r, wlg, wlm, blr)


# speedup vs baseline: 1.3841x; 1.3822x over previous
import functools

import jax
import jax.numpy as jnp
from jax import lax
from jax.experimental import pallas as pl
from jax.experimental.pallas import tpu as pltpu
from jax.experimental.pallas import tpu_sc as plsc

_NUM_CORES = 2
_NUM_SUBCORES = 16
_NW = _NUM_CORES * _NUM_SUBCORES
_C = 128
_PAD = 128


def _sc_gather_t(uidx, iidx, ug_t, um_t, ig_t, im_t):
    b = uidx.shape[0]
    d = ug_t.shape[0]
    n = ug_t.shape[1]
    n_pad = -(-n // 128) * 128
    per_w = -(-n // _NW)
    nch = -(-per_w // _C) + 1
    nch += nch % 2
    hmax = b + 16

    mesh = plsc.VectorSubcoreMesh(
        core_axis_name="c", subcore_axis_name="s")
    out_type = [jax.ShapeDtypeStruct((b + _PAD, 2 * d), jnp.float32)] * 4

    @functools.partial(
        pl.kernel,
        out_type=out_type,
        mesh=mesh,
        compiler_params=pltpu.CompilerParams(
            needs_layout_passes=False, disable_bounds_checks=True),
        scratch_types=[
            pltpu.VMEM((b,), jnp.int32),
            pltpu.VMEM((hmax,), jnp.int32),
            pltpu.VMEM((hmax,), jnp.int32),
            pltpu.VMEM((hmax,), jnp.int32),
            pltpu.VMEM((hmax,), jnp.int32),
            pltpu.VMEM((d, _C), jnp.float32),
            pltpu.VMEM((d, _C), jnp.float32),
            pltpu.VMEM((d, _C), jnp.float32),
            pltpu.VMEM((d, _C), jnp.float32),
            pltpu.VMEM((16, 2 * d), jnp.float32),
            pltpu.VMEM((16, 2 * d), jnp.float32),
            pltpu.VMEM((16, 2 * d), jnp.float32),
            pltpu.VMEM((16, 2 * d), jnp.float32),
            pltpu.SemaphoreType.DMA,
            pltpu.SemaphoreType.DMA,
            pltpu.SemaphoreType.DMA,
            pltpu.SemaphoreType.DMA,
        ],
    )
    def gather_kernel(uidx_hbm, iidx_hbm, ug_hbm, um_hbm, ig_hbm, im_hbm,
                      ug_out, um_out, ig_out, im_out,
                      ids_v, hit_n, hit_j, act_n, act_j,
                      cb00, cb01, cb10, cb11, ro00, ro01, ro10, ro11,
                      sem_c0, sem_c1, sem_s0, sem_s1):
        wid = lax.axis_index("s") * _NUM_CORES + lax.axis_index("c")
        lo = (wid * per_w) // 128 * 128
        csems = (sem_c0, sem_c1)
        ssems = (sem_s0, sem_s1)
        cbufs = ((cb00, cb01), (cb10, cb11))
        rbufs = ((ro00, ro01), (ro10, ro11))

        def scan_hits(idx_hbm):
            pltpu.sync_copy(idx_hbm, ids_v)
            lo_v = jnp.broadcast_to(lo, (16,))
            hi_v = jnp.broadcast_to(lo + nch * _C, (16,))

            def blk(t, off):
                v = ids_v[pl.ds(t * 16, 16)]
                m = (v >= lo_v) & (v < hi_v)
                plsc.store_compressed(hit_n.at[pl.ds(off, 16)], v, mask=m)
                plsc.store_compressed(
                    hit_j.at[pl.ds(off, 16)],
                    lax.iota(jnp.int32, 16) + t * 16, mask=m)
                return off + plsc.all_reduce_population_count(m)[0]

            nh = lax.fori_loop(0, b // 16, blk, jnp.int32(0))
            hit_n[pl.ds(nh, 16)] = jnp.broadcast_to(
                jnp.int32(-1), (16,))
            return nh

        def c_start(c):
            c0 = lo + c * _C
            return jnp.minimum(c0, n_pad - _C)

        def fire(c, phase, tbls):
            c0 = pl.multiple_of(c_start(c), 128)
            for u, tbl in enumerate(tbls):
                pltpu.async_copy(
                    tbl.at[:, pl.ds(c0, _C)], cbufs[phase][u],
                    csems[phase])

        def drain_chunk(phase, tbls):
            for u, tbl in enumerate(tbls):
                pltpu.make_async_copy(
                    tbl.at[:, pl.ds(0, _C)], cbufs[phase][u],
                    csems[phase]).wait()

        def drain_scats(phase, outs, cnt):
            def one(_, carry):
                pltpu.make_async_copy(
                    outs[0].at[pl.ds(0, 16)], rbufs[phase][0],
                    ssems[phase]).wait()
                return carry
            lax.fori_loop(0, cnt, one, jnp.int32(0))

        def process(c, phase, nh, outs):
            c0 = c_start(c)
            lo_c = jnp.broadcast_to(lo + c * _C, (16,))
            hi_c = jnp.broadcast_to(
                jnp.minimum(lo + (c + 1) * _C, n), (16,))

            def blk(t, off):
                v = hit_n[pl.ds(t * 16, 16)]
                m = (v >= lo_c) & (v < hi_c)
                plsc.store_compressed(act_n.at[pl.ds(off, 16)], v, mask=m)
                plsc.store_compressed(
                    act_j.at[pl.ds(off, 16)],
                    hit_j[pl.ds(t * 16, 16)], mask=m)
                return off + plsc.all_reduce_population_count(m)[0]

            nblk = (nh + 15) // 16
            ka = lax.fori_loop(0, nblk, blk, jnp.int32(0))

            def grp(g, cnt):
                @pl.when(g > 0)
                def _():
                    for u in range(2):
                        pltpu.make_async_copy(
                            outs[u].at[pl.ds(0, 16)], rbufs[phase][u],
                            ssems[phase]).wait()
                av = act_n[pl.ds(g * 16, 16)]
                aj = act_j[pl.ds(g * 16, 16)]
                for q in range(16):
                    col = jnp.clip(av[q] - c0, 0, _C - 1)
                    colv = jnp.broadcast_to(col, (16,)).astype(jnp.int32)
                    for u in range(2):
                        for s in range(d // 16):
                            rr = lax.iota(jnp.int32, 16) + 16 * s
                            val = plsc.load_gather(
                                cbufs[phase][u], [rr, colv])
                            rbufs[phase][u][q, pl.ds(16 * s, 16)] = val
                rem = jnp.broadcast_to(ka - g * 16, (16,))
                jfin = jnp.where(
                    lax.iota(jnp.int32, 16) < rem,
                    aj, b + lax.iota(jnp.int32, 16))
                for u in range(2):
                    pltpu.async_copy(
                        rbufs[phase][u], outs[u].at[jfin], ssems[phase])
                return jnp.int32(2)

            ngrp = (ka + 15) // 16
            return lax.fori_loop(0, ngrp, grp, jnp.int32(0))

        def sweep(idx_hbm, tbls, outs):
            nh = scan_hits(idx_hbm)
            fire(0, 0, tbls)
            fire(1, 1, tbls)

            def body(i, cnts):
                cnt0, cnt1 = cnts
                c0, c1 = 2 * i, 2 * i + 1
                drain_chunk(0, tbls)
                drain_scats(0, outs, cnt0)
                cnt0 = process(c0, 0, nh, outs)
                fire(jnp.minimum(c1 + 1, nch - 1), 0, tbls)
                drain_chunk(1, tbls)
                drain_scats(1, outs, cnt1)
                cnt1 = process(c1, 1, nh, outs)
                fire(jnp.minimum(c1 + 2, nch - 1), 1, tbls)
                return (cnt0, cnt1)

            cnt0, cnt1 = lax.fori_loop(
                0, nch // 2, body, (jnp.int32(0), jnp.int32(0)))
            drain_chunk(0, tbls)
            drain_scats(0, outs, cnt0)
            drain_chunk(1, tbls)
            drain_scats(1, outs, cnt1)

        sweep(uidx_hbm, (ug_hbm, um_hbm), (ug_out, um_out))
        sweep(iidx_hbm, (ig_hbm, im_hbm), (ig_out, im_out))

    return gather_kernel(uidx, iidx, ug_t, um_t, ig_t, im_t)


def _mlp_body(ug, ig, um, im, w1u, w1i, b1, w2t, b2, wlg, wlm, bl, out):
    d = w2t.shape[0]
    gmf = ug[:, :d] * ig[:, :d]
    h = (jnp.dot(um[:, :d], w1u[:], preferred_element_type=jnp.float32)
         + jnp.dot(im[:, :d], w1i[:], preferred_element_type=jnp.float32)
         + b1[:])
    h = jnp.maximum(h, 0.0)
    mlp = jnp.dot(h, w2t[:], preferred_element_type=jnp.float32) + b2[:]
    z = (jnp.dot(gmf, wlg[:], preferred_element_type=jnp.float32)
         + jnp.dot(mlp, wlm[:], preferred_element_type=jnp.float32)
         + bl[:])
    out[:] = jax.nn.sigmoid(z)


def kernel(user_ids, item_ids, U_gmf, I_gmf, U_mlp, I_mlp,
           W1, b1, W2, b2, Wl, bl):
    b = user_ids.shape[0]
    d = U_gmf.shape[1]
    uids = user_ids.astype(jnp.int32)
    iids = item_ids.astype(jnp.int32)

    ug, um, ig, im = _sc_gather_t(
        uids, iids, U_gmf.T, U_mlp.T, I_gmf.T, I_mlp.T)

    w1u = W1[:, :d].T
    w1i = W1[:, d:].T
    w2t = W2.T
    wlg = Wl[0, :d].reshape(d, 1)
    wlm = Wl[0, d:].reshape(d, 1)
    b1r = b1.reshape(1, d)
    b2r = b2.reshape(1, d)
    blr = bl.reshape(1, 1)

    bb = 2048
    grid = (b // bb,)
    row_spec = pl.BlockSpec((bb, 2 * d), lambda i: (i, 0))
    full = lambda shape: pl.BlockSpec(shape, lambda i: (0, 0))

    return pl.pallas_call(
        _mlp_body,
        grid=grid,
        in_specs=[
            row_spec, row_spec, row_spec, row_spec,
            full((d, d)), full((d, d)), full((1, d)),
            full((d, d)), full((1, d)),
            full((d, 1)), full((d, 1)), full((1, 1)),
        ],
        out_specs=pl.BlockSpec((bb, 1), lambda i: (i, 0)),
        out_shape=jax.ShapeDtypeStruct((b, 1), jnp.float32),
    )(ug, ig, um, im, w1u, w1i, b1r, w2t, b2r, wlg, wlm, blr)


# speedup vs baseline: 1.9223x; 1.3888x over previous
import functools

import jax
import jax.numpy as jnp
from jax import lax
from jax.experimental import pallas as pl
from jax.experimental.pallas import tpu as pltpu
from jax.experimental.pallas import tpu_sc as plsc

_NUM_CORES = 2
_NUM_SUBCORES = 16
_NW = _NUM_CORES * _NUM_SUBCORES
_C = 256
_PAD = 128


def _sc_gather_t(uidx, iidx, ug_t, um_t, ig_t, im_t):
    b = uidx.shape[0]
    d = ug_t.shape[0]
    n = ug_t.shape[1]
    n_pad = -(-n // 128) * 128
    per_w = -(-n // _NW)
    nch = -(-per_w // _C) + 1
    nch += nch % 2
    hmax = b + 16

    mesh = plsc.VectorSubcoreMesh(
        core_axis_name="c", subcore_axis_name="s")
    out_type = [jax.ShapeDtypeStruct((b + _PAD, 2 * d), jnp.float32)] * 4

    @functools.partial(
        pl.kernel,
        out_type=out_type,
        mesh=mesh,
        compiler_params=pltpu.CompilerParams(
            needs_layout_passes=False, disable_bounds_checks=True),
        scratch_types=[
            pltpu.VMEM((b,), jnp.int32),
            pltpu.VMEM((hmax,), jnp.int32),
            pltpu.VMEM((hmax,), jnp.int32),
            pltpu.VMEM((d, _C), jnp.float32),
            pltpu.VMEM((d, _C), jnp.float32),
            pltpu.VMEM((d, _C), jnp.float32),
            pltpu.VMEM((d, _C), jnp.float32),
            pltpu.VMEM((16, 2 * d), jnp.float32),
            pltpu.VMEM((16, 2 * d), jnp.float32),
            pltpu.VMEM((16, 2 * d), jnp.float32),
            pltpu.VMEM((16, 2 * d), jnp.float32),
            pltpu.SemaphoreType.DMA,
            pltpu.SemaphoreType.DMA,
            pltpu.SemaphoreType.DMA,
            pltpu.SemaphoreType.DMA,
        ],
    )
    def gather_kernel(uidx_hbm, iidx_hbm, ug_hbm, um_hbm, ig_hbm, im_hbm,
                      ug_out, um_out, ig_out, im_out,
                      ids_v, hit_j, act_j,
                      cb00, cb01, cb10, cb11, ro00, ro01, ro10, ro11,
                      sem_c0, sem_c1, sem_s0, sem_s1):
        wid = lax.axis_index("s") * _NUM_CORES + lax.axis_index("c")
        lo = (wid * per_w) // 128 * 128
        csems = (sem_c0, sem_c1)
        ssems = (sem_s0, sem_s1)
        cbufs = ((cb00, cb01), (cb10, cb11))
        rbufs = ((ro00, ro01), (ro10, ro11))
        iota = lax.iota(jnp.int32, 16)

        def scan_hits(idx_hbm):
            pltpu.sync_copy(idx_hbm, ids_v)
            lo_v = jnp.broadcast_to(lo, (16,))
            hi_v = jnp.broadcast_to(lo + nch * _C, (16,))

            def blk(t, off):
                v = ids_v[pl.ds(t * 16, 16)]
                m = (v >= lo_v) & (v < hi_v)
                k = plsc.all_reduce_population_count(m)[0]

                @pl.when(k > 0)
                def _():
                    plsc.store_compressed(
                        hit_j.at[pl.ds(off, 16)], iota + t * 16, mask=m)
                return off + k

            nh = lax.fori_loop(0, b // 16, blk, jnp.int32(0))
            hit_j[pl.ds(nh, 16)] = jnp.broadcast_to(jnp.int32(0), (16,))
            return nh

        def c_start(c):
            return jnp.minimum(lo + c * _C, n_pad - _C)

        def fire(c, phase, tbls):
            c0 = pl.multiple_of(c_start(c), 128)
            for u, tbl in enumerate(tbls):
                pltpu.async_copy(
                    tbl.at[:, pl.ds(c0, _C)], cbufs[phase][u],
                    csems[phase])

        def drain_chunk(phase, tbls):
            for u, tbl in enumerate(tbls):
                pltpu.make_async_copy(
                    tbl.at[:, pl.ds(0, _C)], cbufs[phase][u],
                    csems[phase]).wait()

        def drain_scats(phase, outs, cnt):
            def one(_, carry):
                pltpu.make_async_copy(
                    outs[0].at[pl.ds(0, 16)], rbufs[phase][0],
                    ssems[phase]).wait()
                return carry
            lax.fori_loop(0, cnt, one, jnp.int32(0))

        def process(c, phase, nh, outs):
            c0 = c_start(c)
            lo_c = jnp.broadcast_to(lo + c * _C, (16,))
            hi_c = jnp.broadcast_to(
                jnp.minimum(lo + (c + 1) * _C, n), (16,))

            def blk(t, off):
                jv = hit_j[pl.ds(t * 16, 16)]
                v = plsc.load_gather(ids_v, [jv])
                m = (v >= lo_c) & (v < hi_c)
                k = plsc.all_reduce_population_count(m)[0]

                @pl.when(k > 0)
                def _():
                    plsc.store_compressed(
                        act_j.at[pl.ds(off, 16)], jv, mask=m)
                return off + k

            nblk = (nh + 15) // 16
            ka = lax.fori_loop(0, nblk, blk, jnp.int32(0))
            act_j[pl.ds(ka, 16)] = jnp.broadcast_to(jnp.int32(0), (16,))

            def grp(g, cnt):
                @pl.when(g > 0)
                def _():
                    for u in range(2):
                        pltpu.make_async_copy(
                            outs[u].at[pl.ds(0, 16)], rbufs[phase][u],
                            ssems[phase]).wait()
                aj = act_j[pl.ds(g * 16, 16)]
                av = plsc.load_gather(ids_v, [aj])
                for q in range(16):
                    col = jnp.clip(av[q] - c0, 0, _C - 1)
                    colv = jnp.broadcast_to(col, (16,)).astype(jnp.int32)
                    for u in range(2):
                        for s in range(d // 16):
                            rr = iota + 16 * s
                            val = plsc.load_gather(
                                cbufs[phase][u], [rr, colv])
                            rbufs[phase][u][q, pl.ds(16 * s, 16)] = val
                rem = jnp.broadcast_to(ka - g * 16, (16,))
                jfin = jnp.where(iota < rem, aj, b + iota)
                for u in range(2):
                    pltpu.async_copy(
                        rbufs[phase][u], outs[u].at[jfin], ssems[phase])
                return jnp.int32(2)

            ngrp = (ka + 15) // 16
            return lax.fori_loop(0, ngrp, grp, jnp.int32(0))

        def sweep(idx_hbm, tbls, outs):
            nh = scan_hits(idx_hbm)
            fire(0, 0, tbls)
            fire(1, 1, tbls)

            def body(i, cnts):
                cnt0, cnt1 = cnts
                drain_chunk(0, tbls)
                drain_scats(0, outs, cnt0)
                cnt0 = process(2 * i, 0, nh, outs)
                fire(jnp.minimum(2 * i + 2, nch - 1), 0, tbls)
                drain_chunk(1, tbls)
                drain_scats(1, outs, cnt1)
                cnt1 = process(2 * i + 1, 1, nh, outs)
                fire(jnp.minimum(2 * i + 3, nch - 1), 1, tbls)
                return (cnt0, cnt1)

            cnt0, cnt1 = lax.fori_loop(
                0, nch // 2, body, (jnp.int32(0), jnp.int32(0)))
            drain_chunk(0, tbls)
            drain_scats(0, outs, cnt0)
            drain_chunk(1, tbls)
            drain_scats(1, outs, cnt1)

        sweep(uidx_hbm, (ug_hbm, um_hbm), (ug_out, um_out))
        sweep(iidx_hbm, (ig_hbm, im_hbm), (ig_out, im_out))

    return gather_kernel(uidx, iidx, ug_t, um_t, ig_t, im_t)


def _mlp_body(ug, ig, um, im, w1u, w1i, b1, w2t, b2, wlg, wlm, bl, out):
    d = w2t.shape[0]
    gmf = ug[:, :d] * ig[:, :d]
    h = (jnp.dot(um[:, :d], w1u[:], preferred_element_type=jnp.float32)
         + jnp.dot(im[:, :d], w1i[:], preferred_element_type=jnp.float32)
         + b1[:])
    h = jnp.maximum(h, 0.0)
    mlp = jnp.dot(h, w2t[:], preferred_element_type=jnp.float32) + b2[:]
    z = (jnp.dot(gmf, wlg[:], preferred_element_type=jnp.float32)
         + jnp.dot(mlp, wlm[:], preferred_element_type=jnp.float32)
         + bl[:])
    out[:] = jax.nn.sigmoid(z)


def kernel(user_ids, item_ids, U_gmf, I_gmf, U_mlp, I_mlp,
           W1, b1, W2, b2, Wl, bl):
    b = user_ids.shape[0]
    d = U_gmf.shape[1]
    uids = user_ids.astype(jnp.int32)
    iids = item_ids.astype(jnp.int32)

    ug, um, ig, im = _sc_gather_t(
        uids, iids, U_gmf.T, U_mlp.T, I_gmf.T, I_mlp.T)

    w1u = W1[:, :d].T
    w1i = W1[:, d:].T
    w2t = W2.T
    wlg = Wl[0, :d].reshape(d, 1)
    wlm = Wl[0, d:].reshape(d, 1)
    b1r = b1.reshape(1, d)
    b2r = b2.reshape(1, d)
    blr = bl.reshape(1, 1)

    bb = 2048
    grid = (b // bb,)
    row_spec = pl.BlockSpec((bb, 2 * d), lambda i: (i, 0))
    full = lambda shape: pl.BlockSpec(shape, lambda i: (0, 0))

    return pl.pallas_call(
        _mlp_body,
        grid=grid,
        in_specs=[
            row_spec, row_spec, row_spec, row_spec,
            full((d, d)), full((d, d)), full((1, d)),
            full((d, d)), full((1, d)),
            full((d, 1)), full((d, 1)), full((1, 1)),
        ],
        out_specs=pl.BlockSpec((bb, 1), lambda i: (i, 0)),
        out_shape=jax.ShapeDtypeStruct((b, 1), jnp.float32),
    )(ug, ig, um, im, w1u, w1i, b1r, w2t, b2r, wlg, wlm, blr)
